# Initial kernel scaffold; baseline (speedup 1.0000x reference)
import jax
import jax.numpy as jnp
from jax.experimental import pallas as pl


def kernel(x, W, b):
    raise NotImplementedError("write your pallas kernel here")


# speedup vs baseline: 2.2743x; 2.2743x over previous
import functools

import jax
import jax.numpy as jnp
from jax.experimental import pallas as pl
from jax.experimental.pallas import tpu as pltpu

DIM = 4096
NUM_EXPERTS = 64
EPSILON = 1e-06
BLOCK_N = 512


def _gate_block(x_ref, w_ref, b_ref, masked_ref, denom_ref):
    i = pl.program_id(0)
    logits = jax.lax.dot_general(
        x_ref[:], w_ref[:], (((1,), (1,)), ((), ())),
        preferred_element_type=jnp.float32) + b_ref[:]
    m = jnp.max(logits, axis=1, keepdims=True)
    e = jnp.exp(logits - m)
    s = jnp.sum(e, axis=1, keepdims=True)
    p = e / s
    idx = jnp.argmax(logits, axis=1)
    cols = jax.lax.broadcasted_iota(jnp.int32, p.shape, 1)
    masked = jnp.where(cols == idx[:, None], p, 0.0)
    masked_ref[:] = masked
    part = jnp.sum(masked, axis=0, keepdims=True)

    @pl.when(i == 0)
    def _init():
        denom_ref[:] = part

    @pl.when(i > 0)
    def _acc():
        denom_ref[:] += part


def _normalize(masked_ref, denom_ref, out_ref, *, capacity):
    out_ref[:] = masked_ref[:] * (capacity / (denom_ref[:] + EPSILON))


def kernel(x, W, b):
    batch_size, seq_len, dim = x.shape
    n = batch_size * seq_len
    capacity = float(n)
    xf = x.reshape(n, dim)
    b2 = b.reshape(1, NUM_EXPERTS)
    nblk = n // BLOCK_N

    masked, denom = pl.pallas_call(
        _gate_block,
        grid=(nblk,),
        in_specs=[
            pl.BlockSpec((BLOCK_N, dim), lambda i: (i, 0)),
            pl.BlockSpec((NUM_EXPERTS, dim), lambda i: (0, 0)),
            pl.BlockSpec((1, NUM_EXPERTS), lambda i: (0, 0)),
        ],
        out_specs=[
            pl.BlockSpec((BLOCK_N, NUM_EXPERTS), lambda i: (i, 0)),
            pl.BlockSpec((1, NUM_EXPERTS), lambda i: (0, 0)),
        ],
        out_shape=[
            jax.ShapeDtypeStruct((n, NUM_EXPERTS), jnp.float32),
            jax.ShapeDtypeStruct((1, NUM_EXPERTS), jnp.float32),
        ],
    )(xf, W, b2)

    out = pl.pallas_call(
        functools.partial(_normalize, capacity=capacity),
        in_specs=[
            pl.BlockSpec((n, NUM_EXPERTS), lambda: (0, 0)),
            pl.BlockSpec((1, NUM_EXPERTS), lambda: (0, 0)),
        ],
        out_specs=pl.BlockSpec((n, NUM_EXPERTS), lambda: (0, 0)),
        out_shape=jax.ShapeDtypeStruct((n, NUM_EXPERTS), jnp.float32),
    )(masked, denom)

    return out.reshape(batch_size, seq_len, NUM_EXPERTS)


# speedup vs baseline: 2.4466x; 1.0757x over previous
import jax
import jax.numpy as jnp
from jax.experimental import pallas as pl
from jax.experimental.pallas import tpu as pltpu

DIM = 4096
NUM_EXPERTS = 64
EPSILON = 1e-06
BLOCK_N = 512


def _gate_block(x_ref, w_ref, b_ref, out_ref, masked_ref, denom_ref):
    i = pl.program_id(0)
    nblk = pl.num_programs(0)
    logits = jax.lax.dot_general(
        x_ref[:], w_ref[:], (((1,), (1,)), ((), ())),
        preferred_element_type=jnp.float32) + b_ref[:]
    m = jnp.max(logits, axis=1, keepdims=True)
    e = jnp.exp(logits - m)
    s = jnp.sum(e, axis=1, keepdims=True)
    idx = jnp.argmax(logits, axis=1)
    cols = jax.lax.broadcasted_iota(jnp.int32, logits.shape, 1)
    masked = jnp.where(cols == idx[:, None], 1.0 / s, 0.0)
    masked_ref[pl.ds(i * BLOCK_N, BLOCK_N), :] = masked
    part = jnp.sum(masked, axis=0, keepdims=True)

    @pl.when(i == 0)
    def _init():
        denom_ref[:] = part

    @pl.when(i > 0)
    def _acc():
        denom_ref[:] += part

    @pl.when(i == nblk - 1)
    def _finish():
        n = masked_ref.shape[0]
        scale = float(n) / (denom_ref[:] + EPSILON)
        out_ref[:] = masked_ref[:] * scale


def kernel(x, W, b):
    batch_size, seq_len, dim = x.shape
    n = batch_size * seq_len
    xf = x.reshape(n, dim)
    b2 = b.reshape(1, NUM_EXPERTS)
    nblk = n // BLOCK_N

    out = pl.pallas_call(
        _gate_block,
        grid=(nblk,),
        in_specs=[
            pl.BlockSpec((BLOCK_N, dim), lambda i: (i, 0)),
            pl.BlockSpec((NUM_EXPERTS, dim), lambda i: (0, 0)),
            pl.BlockSpec((1, NUM_EXPERTS), lambda i: (0, 0)),
        ],
        out_specs=pl.BlockSpec((n, NUM_EXPERTS), lambda i: (0, 0)),
        out_shape=jax.ShapeDtypeStruct((n, NUM_EXPERTS), jnp.float32),
        scratch_shapes=[
            pltpu.VMEM((n, NUM_EXPERTS), jnp.float32),
            pltpu.VMEM((1, NUM_EXPERTS), jnp.float32),
        ],
    )(xf, W, b2)

    return out.reshape(batch_size, seq_len, NUM_EXPERTS)
